# baseline (device time: 83274 ns/iter reference)
import jax
import jax.numpy as jnp
from jax import lax
from jax.experimental import pallas as pl
from jax.experimental.pallas import tpu as pltpu

N_DEV = 8
SQ = 256
SKV_LOCAL = 4096
HQ = 8
DH = 128
DM = 1024
BLK = 64
NBLK = SKV_LOCAL // BLK
GCAP = 22
GROWS = GCAP * BLK
SCALE = 0.08838834764831843
LOG2E = 1.4426950408889634
NEG = -1e30
ROUNDS = 3

BF = jnp.bfloat16
F32 = jnp.float32


def kernel(x, Wq, K_ext, V_ext, Wo):
    xq = x.reshape(SQ, DM)
    k = K_ext.reshape(SKV_LOCAL, HQ * DH)
    v = V_ext.reshape(SKV_LOCAL, HQ * DH)

    def body(
        x_ref,
        wq_ref,
        k_ref,
        v_ref,
        wo_ref,
        out_ref,
        kg,
        vg,
        ke,
        ve,
        gsem,
        usend,
        urecv,
        lsend,
        lrecv,
        usend_sems,
        urecv_sems,
        lsend_sems,
        lrecv_sems,
    ):
        my = lax.axis_index("i")

        copies = []
        for r in range(3):
            b0 = (r - my) % 3
            for j in range(GCAP):
                blk = jnp.minimum(b0 + 3 * j, NBLK - 1)
                for src, dst in ((k_ref, kg), (v_ref, vg)):
                    c = pltpu.make_async_copy(
                        src.at[pl.ds(blk * BLK, BLK), :],
                        dst.at[r, pl.ds(j * BLK, BLK), :],
                        gsem,
                    )
                    c.start()
                    copies.append(c)
        for src, dst in ((k_ref, ke), (v_ref, ve)):
            for idx, (ba, bb) in enumerate(((0, 1), (0, 2))):
                for slot, blk in enumerate((ba, bb)):
                    c = pltpu.make_async_copy(
                        src.at[pl.ds(blk * BLK, BLK), :],
                        dst.at[idx, pl.ds(slot * BLK, BLK), :],
                        gsem,
                    )
                    c.start()
                    copies.append(c)

        barrier = pltpu.get_barrier_semaphore()
        for r in range(ROUNDS):
            pl.semaphore_signal(
                barrier,
                inc=1,
                device_id=(my ^ (1 << r),),
                device_id_type=pl.DeviceIdType.MESH,
            )
        pl.semaphore_wait(barrier, ROUNDS)

        q = jnp.dot(
            x_ref[...].astype(BF),
            wq_ref[...].astype(BF),
            preferred_element_type=F32,
        )
        qv = (q * (SCALE * LOG2E)).astype(BF)

        col = lax.broadcasted_iota(jnp.int32, (1, GROWS), 1)
        gbias = []
        for r in range(3):
            valid = jnp.where(r == my % 3, GROWS, GROWS - BLK)
            gbias.append(jnp.where(col < valid, 0.0, NEG))
        ebias = jnp.where(my == 0, 0.0, NEG)

        for cpy in copies:
            cpy.wait()

        q_by_class = (
            jnp.concatenate([qv[0:BLK], qv[3 * BLK : 4 * BLK]], axis=0),
            qv[2 * BLK : 3 * BLK],
            qv[BLK : 2 * BLK],
        )
        u_cls = [[], [], []]
        l_cls = [[], [], []]
        for r in range(3):
            qr = q_by_class[r]
            extra = None if r == 0 else (ke, ve, 2 - r)
            for h in range(HQ):
                sl = slice(h * DH, (h + 1) * DH)
                s = lax.dot_general(
                    qr[:, sl],
                    kg[r, :, sl].astype(BF),
                    (((1,), (1,)), ((), ())),
                    preferred_element_type=F32,
                )
                w = jnp.exp2(s + gbias[r])
                u = jnp.dot(
                    w.astype(BF),
                    vg[r, :, sl].astype(BF),
                    preferred_element_type=F32,
                )
                l = jnp.sum(w, axis=1, keepdims=True)
                if extra is not None:
                    keb, veb, slot = extra
                    se = lax.dot_general(
                        qr[:, sl],
                        keb[slot, :, sl].astype(BF),
                        (((1,), (1,)), ((), ())),
                        preferred_element_type=F32,
                    )
                    we = jnp.exp2(se + ebias)
                    u = u + jnp.dot(
                        we.astype(BF),
                        veb[slot, :, sl].astype(BF),
                        preferred_element_type=F32,
                    )
                    l = l + jnp.sum(we, axis=1, keepdims=True)
                u_cls[r].append(u)
                l_cls[r].append(l)

        u_parts = []
        l_parts = []
        for h in range(HQ):
            u_parts.append(
                jnp.concatenate(
                    [
                        u_cls[0][h][0:BLK],
                        u_cls[2][h],
                        u_cls[1][h],
                        u_cls[0][h][BLK : 2 * BLK],
                    ],
                    axis=0,
                )
            )
            l_parts.append(
                jnp.concatenate(
                    [
                        l_cls[0][h][0:BLK],
                        l_cls[2][h],
                        l_cls[1][h],
                        l_cls[0][h][BLK : 2 * BLK],
                    ],
                    axis=0,
                )
            )
        u = jnp.concatenate(u_parts, axis=1)
        l = jnp.concatenate(l_parts, axis=1)

        for r in range(ROUNDS):
            partner = my ^ (1 << r)
            usend[r, :, :] = u.astype(BF)
            lsend[r, :, :] = l
            rdma_u = pltpu.make_async_remote_copy(
                src_ref=usend.at[r],
                dst_ref=urecv.at[r],
                send_sem=usend_sems.at[r],
                recv_sem=urecv_sems.at[r],
                device_id=(partner,),
                device_id_type=pl.DeviceIdType.MESH,
            )
            rdma_l = pltpu.make_async_remote_copy(
                src_ref=lsend.at[r],
                dst_ref=lrecv.at[r],
                send_sem=lsend_sems.at[r],
                recv_sem=lrecv_sems.at[r],
                device_id=(partner,),
                device_id_type=pl.DeviceIdType.MESH,
            )
            rdma_u.start()
            rdma_l.start()
            rdma_u.wait()
            rdma_l.wait()
            u = u + urecv[r, :, :].astype(F32)
            l = l + lrecv[r, :, :]

        ctx_parts = []
        for h in range(HQ):
            ctx_parts.append(
                (u[:, h * DH : (h + 1) * DH] / l[:, h : h + 1]).astype(BF)
            )
        ctx = jnp.concatenate(ctx_parts, axis=1)
        out_ref[...] = jnp.dot(
            ctx, wo_ref[...].astype(BF), preferred_element_type=F32
        )

    out = pl.pallas_call(
        body,
        out_shape=jax.ShapeDtypeStruct((SQ, DM), F32),
        in_specs=[
            pl.BlockSpec(memory_space=pltpu.VMEM),
            pl.BlockSpec(memory_space=pltpu.VMEM),
            pl.BlockSpec(memory_space=pl.ANY),
            pl.BlockSpec(memory_space=pl.ANY),
            pl.BlockSpec(memory_space=pltpu.VMEM),
        ],
        out_specs=pl.BlockSpec(memory_space=pltpu.VMEM),
        scratch_shapes=[
            pltpu.VMEM((3, GROWS, DM), F32),
            pltpu.VMEM((3, GROWS, DM), F32),
            pltpu.VMEM((2, 2 * BLK, DM), F32),
            pltpu.VMEM((2, 2 * BLK, DM), F32),
            pltpu.SemaphoreType.DMA,
            pltpu.VMEM((ROUNDS, SQ, DM), BF),
            pltpu.VMEM((ROUNDS, SQ, DM), BF),
            pltpu.VMEM((ROUNDS, SQ, HQ), F32),
            pltpu.VMEM((ROUNDS, SQ, HQ), F32),
            pltpu.SemaphoreType.DMA((ROUNDS,)),
            pltpu.SemaphoreType.DMA((ROUNDS,)),
            pltpu.SemaphoreType.DMA((ROUNDS,)),
            pltpu.SemaphoreType.DMA((ROUNDS,)),
        ],
        compiler_params=pltpu.CompilerParams(
            collective_id=0, vmem_limit_bytes=100 * 1024 * 1024
        ),
    )(xq, Wq, k, v, Wo)
    return out.reshape(1, SQ, DM)


# device time: 57123 ns/iter; 1.4578x vs baseline; 1.4578x over previous
import jax
import jax.numpy as jnp
from jax import lax
from jax.experimental import pallas as pl
from jax.experimental.pallas import tpu as pltpu

N_DEV = 8
SQ = 256
SKV_LOCAL = 4096
HQ = 8
DH = 128
DM = 1024
BLK = 64
NBLK = SKV_LOCAL // BLK
GCAP = 22
GROWS = GCAP * BLK
SCALE = 0.08838834764831843
LOG2E = 1.4426950408889634
NEG = -1e30
ROUNDS = 3

BF = jnp.bfloat16
F32 = jnp.float32


def kernel(x, Wq, K_ext, V_ext, Wo):
    xq = x.reshape(SQ, DM)

    def body(
        x_ref,
        wq_ref,
        k_ref,
        v_ref,
        wo_ref,
        out_ref,
        kg,
        vg,
        ke,
        ve,
        gsem,
        usend,
        urecv,
        lsend,
        lrecv,
        usend_sems,
        urecv_sems,
        lsend_sems,
        lrecv_sems,
    ):
        my = lax.axis_index("i")

        copies = []
        for r in range(3):
            b0 = (r - my) % 3
            for j in range(GCAP):
                blk = jnp.minimum(b0 + 3 * j, NBLK - 1)
                for src, dst in ((k_ref, kg), (v_ref, vg)):
                    for h in range(HQ):
                        c = pltpu.make_async_copy(
                            src.at[0, pl.ds(blk * BLK, BLK), h, :],
                            dst.at[r, pl.ds(j * BLK, BLK), pl.ds(h * DH, DH)],
                            gsem,
                        )
                        c.start()
                        copies.append(c)
        for src, dst in ((k_ref, ke), (v_ref, ve)):
            for idx, (ba, bb) in enumerate(((0, 1), (0, 2))):
                for slot, blk in enumerate((ba, bb)):
                    for h in range(HQ):
                        c = pltpu.make_async_copy(
                            src.at[0, pl.ds(blk * BLK, BLK), h, :],
                            dst.at[idx, pl.ds(slot * BLK, BLK), pl.ds(h * DH, DH)],
                            gsem,
                        )
                        c.start()
                        copies.append(c)

        barrier = pltpu.get_barrier_semaphore()
        for r in range(ROUNDS):
            pl.semaphore_signal(
                barrier,
                inc=1,
                device_id=(my ^ (1 << r),),
                device_id_type=pl.DeviceIdType.MESH,
            )
        pl.semaphore_wait(barrier, ROUNDS)

        q = jnp.dot(
            x_ref[...].astype(BF),
            wq_ref[...].astype(BF),
            preferred_element_type=F32,
        )
        qv = (q * (SCALE * LOG2E)).astype(BF)

        col = lax.broadcasted_iota(jnp.int32, (1, GROWS), 1)
        gbias = []
        for r in range(3):
            valid = jnp.where(r == my % 3, GROWS, GROWS - BLK)
            gbias.append(jnp.where(col < valid, 0.0, NEG))
        ebias = jnp.where(my == 0, 0.0, NEG)

        for cpy in copies:
            cpy.wait()

        q_by_class = (
            jnp.concatenate([qv[0:BLK], qv[3 * BLK : 4 * BLK]], axis=0),
            qv[2 * BLK : 3 * BLK],
            qv[BLK : 2 * BLK],
        )
        u_cls = [[], [], []]
        l_cls = [[], [], []]
        for r in range(3):
            qr = q_by_class[r]
            extra = None if r == 0 else (ke, ve, 2 - r)
            for h in range(HQ):
                sl = slice(h * DH, (h + 1) * DH)
                s = lax.dot_general(
                    qr[:, sl],
                    kg[r, :, sl].astype(BF),
                    (((1,), (1,)), ((), ())),
                    preferred_element_type=F32,
                )
                w = jnp.exp2(s + gbias[r])
                u = jnp.dot(
                    w.astype(BF),
                    vg[r, :, sl].astype(BF),
                    preferred_element_type=F32,
                )
                l = jnp.sum(w, axis=1, keepdims=True)
                if extra is not None:
                    keb, veb, slot = extra
                    se = lax.dot_general(
                        qr[:, sl],
                        keb[slot, :, sl].astype(BF),
                        (((1,), (1,)), ((), ())),
                        preferred_element_type=F32,
                    )
                    we = jnp.exp2(se + ebias)
                    u = u + jnp.dot(
                        we.astype(BF),
                        veb[slot, :, sl].astype(BF),
                        preferred_element_type=F32,
                    )
                    l = l + jnp.sum(we, axis=1, keepdims=True)
                u_cls[r].append(u)
                l_cls[r].append(l)

        u_parts = []
        l_parts = []
        for h in range(HQ):
            u_parts.append(
                jnp.concatenate(
                    [
                        u_cls[0][h][0:BLK],
                        u_cls[2][h],
                        u_cls[1][h],
                        u_cls[0][h][BLK : 2 * BLK],
                    ],
                    axis=0,
                )
            )
            l_parts.append(
                jnp.concatenate(
                    [
                        l_cls[0][h][0:BLK],
                        l_cls[2][h],
                        l_cls[1][h],
                        l_cls[0][h][BLK : 2 * BLK],
                    ],
                    axis=0,
                )
            )
        u = jnp.concatenate(u_parts, axis=1)
        l = jnp.concatenate(l_parts, axis=1)

        for r in range(ROUNDS):
            partner = my ^ (1 << r)
            usend[r, :, :] = u.astype(BF)
            lsend[r, :, :] = l
            rdma_u = pltpu.make_async_remote_copy(
                src_ref=usend.at[r],
                dst_ref=urecv.at[r],
                send_sem=usend_sems.at[r],
                recv_sem=urecv_sems.at[r],
                device_id=(partner,),
                device_id_type=pl.DeviceIdType.MESH,
            )
            rdma_l = pltpu.make_async_remote_copy(
                src_ref=lsend.at[r],
                dst_ref=lrecv.at[r],
                send_sem=lsend_sems.at[r],
                recv_sem=lrecv_sems.at[r],
                device_id=(partner,),
                device_id_type=pl.DeviceIdType.MESH,
            )
            rdma_u.start()
            rdma_l.start()
            rdma_u.wait()
            rdma_l.wait()
            u = u + urecv[r, :, :].astype(F32)
            l = l + lrecv[r, :, :]

        ctx_parts = []
        for h in range(HQ):
            ctx_parts.append(
                (u[:, h * DH : (h + 1) * DH] / l[:, h : h + 1]).astype(BF)
            )
        ctx = jnp.concatenate(ctx_parts, axis=1)
        out_ref[...] = jnp.dot(
            ctx, wo_ref[...].astype(BF), preferred_element_type=F32
        )

    out = pl.pallas_call(
        body,
        out_shape=jax.ShapeDtypeStruct((SQ, DM), F32),
        in_specs=[
            pl.BlockSpec(memory_space=pltpu.VMEM),
            pl.BlockSpec(memory_space=pltpu.VMEM),
            pl.BlockSpec(memory_space=pl.ANY),
            pl.BlockSpec(memory_space=pl.ANY),
            pl.BlockSpec(memory_space=pltpu.VMEM),
        ],
        out_specs=pl.BlockSpec(memory_space=pltpu.VMEM),
        scratch_shapes=[
            pltpu.VMEM((3, GROWS, DM), F32),
            pltpu.VMEM((3, GROWS, DM), F32),
            pltpu.VMEM((2, 2 * BLK, DM), F32),
            pltpu.VMEM((2, 2 * BLK, DM), F32),
            pltpu.SemaphoreType.DMA,
            pltpu.VMEM((ROUNDS, SQ, DM), BF),
            pltpu.VMEM((ROUNDS, SQ, DM), BF),
            pltpu.VMEM((ROUNDS, SQ, HQ), F32),
            pltpu.VMEM((ROUNDS, SQ, HQ), F32),
            pltpu.SemaphoreType.DMA((ROUNDS,)),
            pltpu.SemaphoreType.DMA((ROUNDS,)),
            pltpu.SemaphoreType.DMA((ROUNDS,)),
            pltpu.SemaphoreType.DMA((ROUNDS,)),
        ],
        compiler_params=pltpu.CompilerParams(
            collective_id=0, vmem_limit_bytes=100 * 1024 * 1024
        ),
    )(xq, Wq, K_ext, V_ext, Wo)
    return out.reshape(1, SQ, DM)


# device time: 42816 ns/iter; 1.9449x vs baseline; 1.3342x over previous
import jax
import jax.numpy as jnp
from jax import lax
from jax.experimental import pallas as pl
from jax.experimental.pallas import tpu as pltpu

N_DEV = 8
SQ = 256
SKV_LOCAL = 4096
HQ = 8
DH = 128
DM = 1024
BLK = 64
NBLK = SKV_LOCAL // BLK
GCAP = 22
GROWS = GCAP * BLK
SCALE = 0.08838834764831843
LOG2E = 1.4426950408889634
NEG = -1e30
ROUNDS = 3
NCHUNK = 4

BF = jnp.bfloat16
F32 = jnp.float32

CLASS_ORDER = (0, 2, 1)
CLASS_CHUNKS = {0: ((0, 0), (3, BLK)), 2: ((1, 0),), 1: ((2, 0),)}
CHUNK_ORDER = (0, 3, 1, 2)


def kernel(x, Wq, K_ext, V_ext, Wo):
    xq = x.reshape(SQ, DM)

    def body(
        x_ref,
        wq_ref,
        k_ref,
        v_ref,
        wo_ref,
        out_ref,
        kg,
        vg,
        ke,
        ve,
        gsems,
        usend,
        urecv,
        lsend,
        lrecv,
        usend_sems,
        urecv_sems,
        lsend_sems,
        lrecv_sems,
    ):
        my = lax.axis_index("i")

        copies = {0: [], 1: [], 2: []}
        for r in CLASS_ORDER:
            b0 = (r - my) % 3
            for j in range(GCAP):
                blk = jnp.minimum(b0 + 3 * j, NBLK - 1)
                for src, dst in ((k_ref, kg), (v_ref, vg)):
                    for h in range(HQ):
                        c = pltpu.make_async_copy(
                            src.at[0, pl.ds(blk * BLK, BLK), h, :],
                            dst.at[r, pl.ds(j * BLK, BLK), pl.ds(h * DH, DH)],
                            gsems.at[r],
                        )
                        c.start()
                        copies[r].append(c)
        for src, dst in ((k_ref, ke), (v_ref, ve)):
            for slot, (cls, blocks) in enumerate(((2, (0, 1)), (1, (0, 2)))):
                for pos, blk in enumerate(blocks):
                    for h in range(HQ):
                        c = pltpu.make_async_copy(
                            src.at[0, pl.ds(blk * BLK, BLK), h, :],
                            dst.at[slot, pl.ds(pos * BLK, BLK), pl.ds(h * DH, DH)],
                            gsems.at[cls],
                        )
                        c.start()
                        copies[cls].append(c)

        barrier = pltpu.get_barrier_semaphore()
        for r in range(ROUNDS):
            pl.semaphore_signal(
                barrier,
                inc=1,
                device_id=(my ^ (1 << r),),
                device_id_type=pl.DeviceIdType.MESH,
            )
        pl.semaphore_wait(barrier, ROUNDS)

        q = jnp.dot(
            x_ref[...].astype(BF),
            wq_ref[...].astype(BF),
            preferred_element_type=F32,
        )
        qv = (q * (SCALE * LOG2E)).astype(BF)

        col = lax.broadcasted_iota(jnp.int32, (1, GROWS), 1)
        gbias = {}
        for r in range(3):
            valid = jnp.where(r == my % 3, GROWS, GROWS - BLK)
            gbias[r] = jnp.where(col < valid, 0.0, NEG)
        ebias = jnp.where(my == 0, 0.0, NEG)

        q_by_class = {
            0: jnp.concatenate([qv[0:BLK], qv[3 * BLK : 4 * BLK]], axis=0),
            1: qv[2 * BLK : 3 * BLK],
            2: qv[BLK : 2 * BLK],
        }
        extras_slot = {1: 1, 2: 0}

        u_chunk = {}
        l_chunk = {}
        rd_u = {}
        rd_l = {}

        def start_round(c, r):
            partner = my ^ (1 << r)
            usend[c, r, :, :] = u_chunk[c].astype(BF)
            lsend[c, r, :, :] = l_chunk[c]
            rd_u[(c, r)] = pltpu.make_async_remote_copy(
                src_ref=usend.at[c, r],
                dst_ref=urecv.at[c, r],
                send_sem=usend_sems.at[c, r],
                recv_sem=urecv_sems.at[c, r],
                device_id=(partner,),
                device_id_type=pl.DeviceIdType.MESH,
            )
            rd_l[(c, r)] = pltpu.make_async_remote_copy(
                src_ref=lsend.at[c, r],
                dst_ref=lrecv.at[c, r],
                send_sem=lsend_sems.at[c, r],
                recv_sem=lrecv_sems.at[c, r],
                device_id=(partner,),
                device_id_type=pl.DeviceIdType.MESH,
            )
            rd_u[(c, r)].start()
            rd_l[(c, r)].start()

        for cls in CLASS_ORDER:
            for cpy in copies[cls]:
                cpy.wait()
            qr = q_by_class[cls]
            u_heads = []
            l_heads = []
            for h in range(HQ):
                sl = slice(h * DH, (h + 1) * DH)
                s = lax.dot_general(
                    qr[:, sl],
                    kg[cls, :, sl].astype(BF),
                    (((1,), (1,)), ((), ())),
                    preferred_element_type=F32,
                )
                w = jnp.exp2(s + gbias[cls])
                u = jnp.dot(
                    w.astype(BF),
                    vg[cls, :, sl].astype(BF),
                    preferred_element_type=F32,
                )
                l = jnp.sum(w, axis=1, keepdims=True)
                if cls in extras_slot:
                    slot = extras_slot[cls]
                    se = lax.dot_general(
                        qr[:, sl],
                        ke[slot, :, sl].astype(BF),
                        (((1,), (1,)), ((), ())),
                        preferred_element_type=F32,
                    )
                    we = jnp.exp2(se + ebias)
                    u = u + jnp.dot(
                        we.astype(BF),
                        ve[slot, :, sl].astype(BF),
                        preferred_element_type=F32,
                    )
                    l = l + jnp.sum(we, axis=1, keepdims=True)
                u_heads.append(u)
                l_heads.append(l)
            for chunk, row0 in CLASS_CHUNKS[cls]:
                u_chunk[chunk] = jnp.concatenate(
                    [uh[row0 : row0 + BLK] for uh in u_heads], axis=1
                )
                l_chunk[chunk] = jnp.concatenate(
                    [lh[row0 : row0 + BLK] for lh in l_heads], axis=1
                )
                start_round(chunk, 0)

        wo_bf = wo_ref[...].astype(BF)
        for r in range(ROUNDS):
            for c in CHUNK_ORDER:
                rd_u[(c, r)].wait()
                rd_l[(c, r)].wait()
                u_chunk[c] = u_chunk[c] + urecv[c, r, :, :].astype(F32)
                l_chunk[c] = l_chunk[c] + lrecv[c, r, :, :]
                if r < ROUNDS - 1:
                    start_round(c, r + 1)
                else:
                    u = u_chunk[c]
                    l = l_chunk[c]
                    ctx = jnp.concatenate(
                        [
                            (u[:, h * DH : (h + 1) * DH] / l[:, h : h + 1]).astype(BF)
                            for h in range(HQ)
                        ],
                        axis=1,
                    )
                    out_ref[pl.ds(c * BLK, BLK), :] = jnp.dot(
                        ctx, wo_bf, preferred_element_type=F32
                    )

    out = pl.pallas_call(
        body,
        out_shape=jax.ShapeDtypeStruct((SQ, DM), F32),
        in_specs=[
            pl.BlockSpec(memory_space=pltpu.VMEM),
            pl.BlockSpec(memory_space=pltpu.VMEM),
            pl.BlockSpec(memory_space=pl.ANY),
            pl.BlockSpec(memory_space=pl.ANY),
            pl.BlockSpec(memory_space=pltpu.VMEM),
        ],
        out_specs=pl.BlockSpec(memory_space=pltpu.VMEM),
        scratch_shapes=[
            pltpu.VMEM((3, GROWS, DM), F32),
            pltpu.VMEM((3, GROWS, DM), F32),
            pltpu.VMEM((2, 2 * BLK, DM), F32),
            pltpu.VMEM((2, 2 * BLK, DM), F32),
            pltpu.SemaphoreType.DMA((3,)),
            pltpu.VMEM((NCHUNK, ROUNDS, BLK, DM), BF),
            pltpu.VMEM((NCHUNK, ROUNDS, BLK, DM), BF),
            pltpu.VMEM((NCHUNK, ROUNDS, BLK, HQ), F32),
            pltpu.VMEM((NCHUNK, ROUNDS, BLK, HQ), F32),
            pltpu.SemaphoreType.DMA((NCHUNK, ROUNDS)),
            pltpu.SemaphoreType.DMA((NCHUNK, ROUNDS)),
            pltpu.SemaphoreType.DMA((NCHUNK, ROUNDS)),
            pltpu.SemaphoreType.DMA((NCHUNK, ROUNDS)),
        ],
        compiler_params=pltpu.CompilerParams(
            collective_id=0, vmem_limit_bytes=100 * 1024 * 1024
        ),
    )(xq, Wq, K_ext, V_ext, Wo)
    return out.reshape(1, SQ, DM)


# device time: 42807 ns/iter; 1.9453x vs baseline; 1.0002x over previous
import jax
import jax.numpy as jnp
from jax import lax
from jax.experimental import pallas as pl
from jax.experimental.pallas import tpu as pltpu

N_DEV = 8
SQ = 256
SKV_LOCAL = 4096
HQ = 8
DH = 128
DM = 1024
BLK = 64
NBLK = SKV_LOCAL // BLK
GCAP = 22
GROWS = GCAP * BLK
SCALE = 0.08838834764831843
LOG2E = 1.4426950408889634
NEG = -1e30
ROUNDS = 3
NCHUNK = 4

BF = jnp.bfloat16
F32 = jnp.float32

CLASS_ORDER = (0, 2, 1)
CLASS_CHUNKS = {0: ((0, 0), (3, BLK)), 2: ((1, 0),), 1: ((2, 0),)}
CHUNK_ORDER = (0, 3, 1, 2)


def kernel(x, Wq, K_ext, V_ext, Wo):

    def body(
        x_ref,
        wq_ref,
        k_ref,
        v_ref,
        wo_ref,
        out_ref,
        kg,
        vg,
        ke,
        ve,
        gsems,
        usend,
        urecv,
        lsend,
        lrecv,
        usend_sems,
        urecv_sems,
        lsend_sems,
        lrecv_sems,
    ):
        my = lax.axis_index("i")

        copies = {0: [], 1: [], 2: []}
        for r in CLASS_ORDER:
            b0 = (r - my) % 3
            for j in range(GCAP):
                blk = jnp.minimum(b0 + 3 * j, NBLK - 1)
                for src, dst in ((k_ref, kg), (v_ref, vg)):
                    for h in range(HQ):
                        c = pltpu.make_async_copy(
                            src.at[0, pl.ds(blk * BLK, BLK), h, :],
                            dst.at[r, pl.ds(j * BLK, BLK), pl.ds(h * DH, DH)],
                            gsems.at[r],
                        )
                        c.start()
                        copies[r].append(c)
        for src, dst in ((k_ref, ke), (v_ref, ve)):
            for slot, (cls, blocks) in enumerate(((2, (0, 1)), (1, (0, 2)))):
                for pos, blk in enumerate(blocks):
                    for h in range(HQ):
                        c = pltpu.make_async_copy(
                            src.at[0, pl.ds(blk * BLK, BLK), h, :],
                            dst.at[slot, pl.ds(pos * BLK, BLK), pl.ds(h * DH, DH)],
                            gsems.at[cls],
                        )
                        c.start()
                        copies[cls].append(c)

        barrier = pltpu.get_barrier_semaphore()
        for r in range(ROUNDS):
            pl.semaphore_signal(
                barrier,
                inc=1,
                device_id=(my ^ (1 << r),),
                device_id_type=pl.DeviceIdType.MESH,
            )
        pl.semaphore_wait(barrier, ROUNDS)

        q = jnp.dot(
            x_ref[0].astype(BF),
            wq_ref[...].astype(BF),
            preferred_element_type=F32,
        )
        qv = (q * (SCALE * LOG2E)).astype(BF)

        col = lax.broadcasted_iota(jnp.int32, (1, GROWS), 1)
        gbias = {}
        for r in range(3):
            valid = jnp.where(r == my % 3, GROWS, GROWS - BLK)
            gbias[r] = jnp.where(col < valid, 0.0, NEG).astype(BF)
        ebias = jnp.where(my == 0, 0.0, NEG).astype(BF)

        q_by_class = {
            0: jnp.concatenate([qv[0:BLK], qv[3 * BLK : 4 * BLK]], axis=0),
            1: qv[2 * BLK : 3 * BLK],
            2: qv[BLK : 2 * BLK],
        }
        extras_slot = {1: 1, 2: 0}

        u_chunk = {}
        l_chunk = {}
        rd_u = {}
        rd_l = {}

        def start_round(c, r):
            partner = my ^ (1 << r)
            usend[c, r, :, :] = u_chunk[c].astype(BF)
            lsend[c, r, :, :] = l_chunk[c]
            rd_u[(c, r)] = pltpu.make_async_remote_copy(
                src_ref=usend.at[c, r],
                dst_ref=urecv.at[c, r],
                send_sem=usend_sems.at[c, r],
                recv_sem=urecv_sems.at[c, r],
                device_id=(partner,),
                device_id_type=pl.DeviceIdType.MESH,
            )
            rd_l[(c, r)] = pltpu.make_async_remote_copy(
                src_ref=lsend.at[c, r],
                dst_ref=lrecv.at[c, r],
                send_sem=lsend_sems.at[c, r],
                recv_sem=lrecv_sems.at[c, r],
                device_id=(partner,),
                device_id_type=pl.DeviceIdType.MESH,
            )
            rd_u[(c, r)].start()
            rd_l[(c, r)].start()

        for cls in CLASS_ORDER:
            for cpy in copies[cls]:
                cpy.wait()
            qr = q_by_class[cls]
            u_heads = []
            l_heads = []
            for h in range(HQ):
                sl = slice(h * DH, (h + 1) * DH)
                s = lax.dot_general(
                    qr[:, sl],
                    kg[cls, :, sl].astype(BF),
                    (((1,), (1,)), ((), ())),
                    preferred_element_type=F32,
                )
                w = jnp.exp2(s.astype(BF) + gbias[cls])
                u = jnp.dot(
                    w,
                    vg[cls, :, sl].astype(BF),
                    preferred_element_type=F32,
                )
                l = jnp.sum(w, axis=1, keepdims=True, dtype=F32)
                if cls in extras_slot:
                    slot = extras_slot[cls]
                    se = lax.dot_general(
                        qr[:, sl],
                        ke[slot, :, sl].astype(BF),
                        (((1,), (1,)), ((), ())),
                        preferred_element_type=F32,
                    )
                    we = jnp.exp2(se.astype(BF) + ebias)
                    u = u + jnp.dot(
                        we,
                        ve[slot, :, sl].astype(BF),
                        preferred_element_type=F32,
                    )
                    l = l + jnp.sum(we, axis=1, keepdims=True, dtype=F32)
                u_heads.append(u)
                l_heads.append(l)
            for chunk, row0 in CLASS_CHUNKS[cls]:
                u_chunk[chunk] = jnp.concatenate(
                    [uh[row0 : row0 + BLK] for uh in u_heads], axis=1
                )
                l_chunk[chunk] = jnp.concatenate(
                    [lh[row0 : row0 + BLK] for lh in l_heads], axis=1
                )
                start_round(chunk, 0)

        wo_bf = wo_ref[...].astype(BF)
        for r in range(ROUNDS):
            for c in CHUNK_ORDER:
                rd_u[(c, r)].wait()
                rd_l[(c, r)].wait()
                u_chunk[c] = u_chunk[c] + urecv[c, r, :, :].astype(F32)
                l_chunk[c] = l_chunk[c] + lrecv[c, r, :, :]
                if r < ROUNDS - 1:
                    start_round(c, r + 1)
                else:
                    u = u_chunk[c]
                    l = l_chunk[c]
                    ctx = jnp.concatenate(
                        [
                            (u[:, h * DH : (h + 1) * DH] / l[:, h : h + 1]).astype(BF)
                            for h in range(HQ)
                        ],
                        axis=1,
                    )
                    out_ref[0, pl.ds(c * BLK, BLK), :] = jnp.dot(
                        ctx, wo_bf, preferred_element_type=F32
                    )

    out = pl.pallas_call(
        body,
        out_shape=jax.ShapeDtypeStruct((1, SQ, DM), F32),
        in_specs=[
            pl.BlockSpec(memory_space=pltpu.VMEM),
            pl.BlockSpec(memory_space=pltpu.VMEM),
            pl.BlockSpec(memory_space=pl.ANY),
            pl.BlockSpec(memory_space=pl.ANY),
            pl.BlockSpec(memory_space=pltpu.VMEM),
        ],
        out_specs=pl.BlockSpec(memory_space=pltpu.VMEM),
        scratch_shapes=[
            pltpu.VMEM((3, GROWS, DM), F32),
            pltpu.VMEM((3, GROWS, DM), F32),
            pltpu.VMEM((2, 2 * BLK, DM), F32),
            pltpu.VMEM((2, 2 * BLK, DM), F32),
            pltpu.SemaphoreType.DMA((3,)),
            pltpu.VMEM((NCHUNK, ROUNDS, BLK, DM), BF),
            pltpu.VMEM((NCHUNK, ROUNDS, BLK, DM), BF),
            pltpu.VMEM((NCHUNK, ROUNDS, BLK, HQ), F32),
            pltpu.VMEM((NCHUNK, ROUNDS, BLK, HQ), F32),
            pltpu.SemaphoreType.DMA((NCHUNK, ROUNDS)),
            pltpu.SemaphoreType.DMA((NCHUNK, ROUNDS)),
            pltpu.SemaphoreType.DMA((NCHUNK, ROUNDS)),
            pltpu.SemaphoreType.DMA((NCHUNK, ROUNDS)),
        ],
        compiler_params=pltpu.CompilerParams(
            collective_id=0, vmem_limit_bytes=100 * 1024 * 1024
        ),
    )(x, Wq, K_ext, V_ext, Wo)
    return out
